# baseline (speedup 1.0000x reference)
import functools

import jax
import jax.numpy as jnp
from jax import lax
from jax.experimental import pallas as pl
from jax.experimental.pallas import tpu as pltpu
from jax.experimental.pallas import tpu_sc as plsc

N_NODES = 10000
NUM_GRAPHS = 64
D = 128
D_OUT = 64

NUM_SC = 2
NUM_TILES = 16
CHUNK = 128
CPT = 80
NB = 2
NPHASE = 2
SPC = CPT // NPHASE
E_PAD = NUM_SC * NUM_TILES * CPT * CHUNK
N_PAD = 10240
ROWS_PER_TILE = N_PAD // NUM_TILES
ROWS = 2000
GRID = N_NODES // ROWS


def _sc_edge_pass(y, src2d, dst2d, zeros_blk, zeros_deg, with_deg):
    mesh = plsc.VectorSubcoreMesh(
        core_axis_name="c", subcore_axis_name="s",
        num_cores=NUM_SC, num_subcores=NUM_TILES)

    out_type = [jax.ShapeDtypeStruct((NUM_SC, N_PAD, D), jnp.float32)]
    if with_deg:
        out_type.append(jax.ShapeDtypeStruct((NUM_SC, N_PAD), jnp.float32))

    scratch = [
        pltpu.VMEM_SHARED((N_PAD, D), jnp.float32),
        pltpu.VMEM((SPC, CHUNK), jnp.int32),
        pltpu.VMEM((SPC, CHUNK), jnp.int32),
        pltpu.VMEM((NB, CHUNK, D), jnp.float32),
        pltpu.SemaphoreType.DMA,
        pltpu.SemaphoreType.DMA,
        pltpu.SemaphoreType.DMA,
        pltpu.SemaphoreType.DMA,
    ]
    if with_deg:
        scratch += [
            pltpu.VMEM_SHARED((N_PAD,), jnp.float32),
            pltpu.VMEM((CHUNK,), jnp.float32),
            pltpu.SemaphoreType.DMA,
        ]

    def body(*refs):
        if with_deg:
            (y_hbm, src_hbm, dst_hbm, zblk_hbm, zdeg_hbm,
             agg_hbm, deg_hbm, accum, src_t, dst_t, rows, g0, g1, s0, s1,
             deg_sh, ones, sem_d) = refs
        else:
            (y_hbm, src_hbm, dst_hbm, zblk_hbm,
             agg_hbm, accum, src_t, dst_t, rows, g0, g1, s0, s1) = refs
        sem_g = (g0, g1)
        sem_s = (s0, s1)
        c = lax.axis_index("c")
        s = lax.axis_index("s")
        r0 = s * ROWS_PER_TILE

        zero_cp = pltpu.async_copy(
            zblk_hbm.at[pl.ds(r0, ROWS_PER_TILE)],
            accum.at[pl.ds(r0, ROWS_PER_TILE)], s0)
        if with_deg:
            dzero_cp = pltpu.async_copy(
                zdeg_hbm.at[pl.ds(r0, ROWS_PER_TILE)],
                deg_sh.at[pl.ds(r0, ROWS_PER_TILE)], s1)
            for j in range(CHUNK // 16):
                ones[pl.ds(j * 16, 16)] = jnp.full((16,), 1.0, jnp.float32)

        row_base = (c * NUM_TILES + s) * CPT
        for p in range(NPHASE):
            src_cp = pltpu.async_copy(
                src_hbm.at[pl.ds(row_base + p * SPC, SPC)], src_t, g0)
            dst_cp = pltpu.async_copy(
                dst_hbm.at[pl.ds(row_base + p * SPC, SPC)], dst_t, g1)
            src_cp.wait()

            for b in range(NB):
                pltpu.async_copy(y_hbm.at[src_t.at[b]], rows.at[b], sem_g[b])

            dst_cp.wait()
            if p == 0:
                zero_cp.wait()
                if with_deg:
                    dzero_cp.wait()
                plsc.subcore_barrier()

            def step(i, carry):
                for b in range(NB):
                    j = i * NB + b
                    idx_d = dst_t.at[j]
                    pltpu.make_async_copy(y_hbm.at[src_t.at[j]], rows.at[b],
                                          sem_g[b]).wait()
                    if with_deg:
                        pltpu.async_copy(ones, deg_sh.at[idx_d], sem_d,
                                         add=True)
                    pltpu.sync_copy(rows.at[b], accum.at[idx_d], add=True)

                    nj = j + NB

                    @pl.when(nj < SPC)
                    def _():
                        pltpu.async_copy(y_hbm.at[src_t.at[nj]], rows.at[b],
                                         sem_g[b])
                return carry

            lax.fori_loop(0, SPC // NB, step, 0)

            if with_deg:
                def drain(i, carry):
                    pltpu.make_async_copy(ones, deg_sh.at[dst_t.at[0]],
                                          sem_d).wait()
                    return carry
                lax.fori_loop(0, SPC, drain, 0)

        plsc.subcore_barrier()

        pltpu.sync_copy(accum.at[pl.ds(r0, ROWS_PER_TILE)],
                        agg_hbm.at[c, pl.ds(r0, ROWS_PER_TILE)])
        if with_deg:
            pltpu.sync_copy(deg_sh.at[pl.ds(r0, ROWS_PER_TILE)],
                            deg_hbm.at[c, pl.ds(r0, ROWS_PER_TILE)])

    run = pl.kernel(body, out_type=out_type, mesh=mesh, scratch_types=scratch)
    if with_deg:
        return run(y, src2d, dst2d, zeros_blk, zeros_deg)
    return run(y, src2d, dst2d, zeros_blk)


def _dotT(a, w):
    return lax.dot_general(a, w, (((1,), (1,)), ((), ())),
                           preferred_element_type=jnp.float32)


def _mm_body(x_ref, w_ref, b_ref, y_ref):
    y_ref[...] = _dotT(x_ref[...], w_ref[...]) + b_ref[...]


def _tc_mm(x, W, b):
    return pl.pallas_call(
        _mm_body,
        grid=(GRID,),
        in_specs=[
            pl.BlockSpec((ROWS, D), lambda i: (i, 0)),
            pl.BlockSpec((D, D), lambda i: (0, 0)),
            pl.BlockSpec((1, D), lambda i: (0, 0)),
        ],
        out_specs=pl.BlockSpec((ROWS, D), lambda i: (i, 0)),
        out_shape=jax.ShapeDtypeStruct((N_NODES, D), jnp.float32),
    )(x, W, b.reshape(1, D))


def _tcB_body(a0_ref, a1_ref, d0_ref, d1_ref, s_ref, w_ref, b_ref, y_ref):
    deg = jnp.maximum(d0_ref[...] + d1_ref[...], 1.0)
    h = jnp.maximum((a0_ref[0] + a1_ref[0]) / deg + s_ref[...], 0.0)
    y_ref[...] = _dotT(h, w_ref[...]) + b_ref[...]


def _tc_mid(agg, d0T, d1T, selfp, W, b):
    return pl.pallas_call(
        _tcB_body,
        grid=(GRID,),
        in_specs=[
            pl.BlockSpec((1, ROWS, D), lambda i: (0, i, 0)),
            pl.BlockSpec((1, ROWS, D), lambda i: (1, i, 0)),
            pl.BlockSpec((ROWS, 1), lambda i: (i, 0)),
            pl.BlockSpec((ROWS, 1), lambda i: (i, 0)),
            pl.BlockSpec((ROWS, D), lambda i: (i, 0)),
            pl.BlockSpec((D, D), lambda i: (0, 0)),
            pl.BlockSpec((1, D), lambda i: (0, 0)),
        ],
        out_specs=pl.BlockSpec((ROWS, D), lambda i: (i, 0)),
        out_shape=jax.ShapeDtypeStruct((N_NODES, D), jnp.float32),
    )(agg, agg, d0T, d1T, selfp, W, b.reshape(1, D))


def _tcC_body(a0_ref, a1_ref, d0_ref, d1_ref, s_ref, bt_ref, wlin_ref,
              blin_ref, out_ref, sums, cnts):
    i = pl.program_id(0)

    @pl.when(i == 0)
    def _():
        sums[...] = jnp.zeros_like(sums)
        cnts[...] = jnp.zeros_like(cnts)

    deg = jnp.maximum(d0_ref[...] + d1_ref[...], 1.0)
    h = jnp.maximum((a0_ref[0] + a1_ref[0]) / deg + s_ref[...], 0.0)
    bt = bt_ref[0]
    oh = (lax.broadcasted_iota(jnp.int32, (NUM_GRAPHS, ROWS), 0)
          == bt).astype(jnp.float32)
    sums[...] += jnp.dot(oh, h, preferred_element_type=jnp.float32)
    cnts[...] += jnp.sum(oh, axis=1, keepdims=True)

    @pl.when(i == pl.num_programs(0) - 1)
    def _():
        pooled = sums[...] / jnp.maximum(cnts[...], 1.0)
        out_ref[...] = _dotT(pooled, wlin_ref[...]) + blin_ref[...]


def _tc_post(agg, d0T, d1T, selfp, batch3, Wlin, blin):
    return pl.pallas_call(
        _tcC_body,
        grid=(GRID,),
        in_specs=[
            pl.BlockSpec((1, ROWS, D), lambda i: (0, i, 0)),
            pl.BlockSpec((1, ROWS, D), lambda i: (1, i, 0)),
            pl.BlockSpec((ROWS, 1), lambda i: (i, 0)),
            pl.BlockSpec((ROWS, 1), lambda i: (i, 0)),
            pl.BlockSpec((ROWS, D), lambda i: (i, 0)),
            pl.BlockSpec((1, 1, ROWS), lambda i: (i, 0, 0)),
            pl.BlockSpec((D_OUT, D), lambda i: (0, 0)),
            pl.BlockSpec((1, D_OUT), lambda i: (0, 0)),
        ],
        out_specs=pl.BlockSpec((NUM_GRAPHS, D_OUT), lambda i: (0, 0)),
        out_shape=jax.ShapeDtypeStruct((NUM_GRAPHS, D_OUT), jnp.float32),
        scratch_shapes=[
            pltpu.VMEM((NUM_GRAPHS, D), jnp.float32),
            pltpu.VMEM((NUM_GRAPHS, 1), jnp.float32),
        ],
    )(agg, agg, d0T, d1T, selfp, batch3, Wlin, blin.reshape(1, D_OUT))


def _degcol(deg_slice):
    return deg_slice[:N_NODES].reshape(N_NODES, 1)


def kernel(x, edge_index, batch, W1l, b1, W1r, W2l, b2, W2r, Wlin, blin):
    E = edge_index.shape[1]
    src = edge_index[0]
    dst = edge_index[1]

    npad = E_PAD - E
    ar = jnp.arange(npad, dtype=jnp.int32)
    src_p = jnp.concatenate([src, (ar * 997) % N_NODES])
    dst_p = jnp.concatenate([dst, N_NODES + ar % (N_PAD - N_NODES)])
    src2d = src_p.reshape(E_PAD // CHUNK, CHUNK)
    dst2d = dst_p.reshape(E_PAD // CHUNK, CHUNK)

    zeros_blk = jnp.zeros((N_PAD, D), jnp.float32)
    zeros_deg = jnp.zeros((N_PAD,), jnp.float32)
    batch3 = batch.reshape(GRID, 1, ROWS)

    zero_b = jnp.zeros((D,), jnp.float32)
    y1 = _tc_mm(x, W1l, zero_b)
    agg1, deg = _sc_edge_pass(y1, src2d, dst2d, zeros_blk, zeros_deg,
                              with_deg=True)
    self1 = _tc_mm(x, W1r, b1)
    d0T = _degcol(deg[0])
    d1T = _degcol(deg[1])

    y2 = _tc_mid(agg1, d0T, d1T, self1, W2l, zero_b)
    (agg2,) = _sc_edge_pass(y2, src2d, dst2d, zeros_blk, zeros_deg,
                            with_deg=False)
    self2 = _tc_mid(agg1, d0T, d1T, self1, W2r, b2)

    return _tc_post(agg2, d0T, d1T, self2, batch3, Wlin, blin)


# speedup vs baseline: 1.0136x; 1.0136x over previous
import jax
import jax.numpy as jnp
import numpy as np
from jax import lax
from jax.experimental import pallas as pl
from jax.experimental.pallas import tpu as pltpu
from jax.experimental.pallas import tpu_sc as plsc

N_NODES = 10000
NUM_GRAPHS = 64
D = 128
D_OUT = 64

NUM_SC = 2
NUM_TILES = 16
CHUNK = 128
CPT = 80
NB = 2
NPHASE = 2
SPC = CPT // NPHASE
E_PAD = NUM_SC * NUM_TILES * CPT * CHUNK
N_PAD = 10240
ROWS_PER_TILE = N_PAD // NUM_TILES
ROWS = 2000
GRID = N_NODES // ROWS


def _sc_edge_pass(y, src2d, dst2d, zeros_blk, zeros_deg, with_deg):
    mesh = plsc.VectorSubcoreMesh(
        core_axis_name="c", subcore_axis_name="s",
        num_cores=NUM_SC, num_subcores=NUM_TILES)

    out_type = [jax.ShapeDtypeStruct((NUM_SC, N_PAD, D), jnp.float32)]
    if with_deg:
        out_type.append(jax.ShapeDtypeStruct((NUM_SC, N_PAD), jnp.float32))

    scratch = [
        pltpu.VMEM_SHARED((N_PAD, D), jnp.float32),
        pltpu.VMEM((SPC, CHUNK), jnp.int32),
        pltpu.VMEM((SPC, CHUNK), jnp.int32),
        pltpu.VMEM((NB, CHUNK, D), jnp.float32),
        pltpu.SemaphoreType.DMA,
        pltpu.SemaphoreType.DMA,
        pltpu.SemaphoreType.DMA,
        pltpu.SemaphoreType.DMA,
    ]
    if with_deg:
        scratch += [
            pltpu.VMEM_SHARED((N_PAD,), jnp.float32),
            pltpu.VMEM((CHUNK,), jnp.float32),
            pltpu.SemaphoreType.DMA,
        ]

    def body(*refs):
        if with_deg:
            (y_hbm, src_hbm, dst_hbm, zblk_hbm, zdeg_hbm,
             agg_hbm, deg_hbm, accum, src_t, dst_t, rows, g0, g1, s0, s1,
             deg_sh, ones, sem_d) = refs
        else:
            (y_hbm, src_hbm, dst_hbm, zblk_hbm,
             agg_hbm, accum, src_t, dst_t, rows, g0, g1, s0, s1) = refs
        sem_g = (g0, g1)
        sem_s = (s0, s1)
        c = lax.axis_index("c")
        s = lax.axis_index("s")
        r0 = s * ROWS_PER_TILE

        zero_cp = pltpu.async_copy(
            zblk_hbm.at[pl.ds(r0, ROWS_PER_TILE)],
            accum.at[pl.ds(r0, ROWS_PER_TILE)], s0)
        if with_deg:
            dzero_cp = pltpu.async_copy(
                zdeg_hbm.at[pl.ds(r0, ROWS_PER_TILE)],
                deg_sh.at[pl.ds(r0, ROWS_PER_TILE)], s1)
            for j in range(CHUNK // 16):
                ones[pl.ds(j * 16, 16)] = jnp.full((16,), 1.0, jnp.float32)

        row_base = (c * NUM_TILES + s) * CPT
        for p in range(NPHASE):
            src_cp = pltpu.async_copy(
                src_hbm.at[pl.ds(row_base + p * SPC, SPC)], src_t, g0)
            dst_cp = pltpu.async_copy(
                dst_hbm.at[pl.ds(row_base + p * SPC, SPC)], dst_t, g1)
            src_cp.wait()

            for b in range(NB):
                pltpu.async_copy(y_hbm.at[src_t.at[b]], rows.at[b], sem_g[b])

            dst_cp.wait()
            if p == 0:
                zero_cp.wait()
                if with_deg:
                    dzero_cp.wait()
                plsc.subcore_barrier()

            def step(i, carry):
                for b in range(NB):
                    j = i * NB + b
                    idx_d = dst_t.at[j]
                    pltpu.make_async_copy(y_hbm.at[src_t.at[j]], rows.at[b],
                                          sem_g[b]).wait()
                    if with_deg:
                        pltpu.async_copy(ones, deg_sh.at[idx_d], sem_d,
                                         add=True)
                    pltpu.sync_copy(rows.at[b], accum.at[idx_d], add=True)

                    nj = j + NB

                    @pl.when(nj < SPC)
                    def _():
                        pltpu.async_copy(y_hbm.at[src_t.at[nj]], rows.at[b],
                                         sem_g[b])
                return carry

            lax.fori_loop(0, SPC // NB, step, 0)

            if with_deg:
                def drain(i, carry):
                    pltpu.make_async_copy(ones, deg_sh.at[dst_t.at[0]],
                                          sem_d).wait()
                    return carry
                lax.fori_loop(0, SPC, drain, 0)

        plsc.subcore_barrier()

        pltpu.sync_copy(accum.at[pl.ds(r0, ROWS_PER_TILE)],
                        agg_hbm.at[c, pl.ds(r0, ROWS_PER_TILE)])
        if with_deg:
            pltpu.sync_copy(deg_sh.at[pl.ds(r0, ROWS_PER_TILE)],
                            deg_hbm.at[c, pl.ds(r0, ROWS_PER_TILE)])

    run = pl.kernel(body, out_type=out_type, mesh=mesh, scratch_types=scratch)
    if with_deg:
        return run(y, src2d, dst2d, zeros_blk, zeros_deg)
    return run(y, src2d, dst2d, zeros_blk)


def _dotT(a, w):
    return lax.dot_general(a, w, (((1,), (1,)), ((), ())),
                           preferred_element_type=jnp.float32)


def _tcA_body(x_ref, wl_ref, wr_ref, b_ref, y_ref, self_ref):
    xb = x_ref[...]
    y_ref[...] = _dotT(xb, wl_ref[...])
    self_ref[...] = _dotT(xb, wr_ref[...]) + b_ref[...]


def _tc_pre(x, Wl, Wr, b):
    return pl.pallas_call(
        _tcA_body,
        grid=(GRID,),
        in_specs=[
            pl.BlockSpec((ROWS, D), lambda i: (i, 0)),
            pl.BlockSpec((D, D), lambda i: (0, 0)),
            pl.BlockSpec((D, D), lambda i: (0, 0)),
            pl.BlockSpec((1, D), lambda i: (0, 0)),
        ],
        out_specs=[
            pl.BlockSpec((ROWS, D), lambda i: (i, 0)),
            pl.BlockSpec((ROWS, D), lambda i: (i, 0)),
        ],
        out_shape=[
            jax.ShapeDtypeStruct((N_NODES, D), jnp.float32),
            jax.ShapeDtypeStruct((N_NODES, D), jnp.float32),
        ],
    )(x, Wl, Wr, b.reshape(1, D))


def _tcB_body(a_ref, d0_ref, d1_ref, s_ref, wl_ref, wr_ref, b_ref,
              y_ref, self_ref):
    deg = jnp.maximum(d0_ref[...] + d1_ref[...], 1.0)
    h = jnp.maximum((a_ref[0] + a_ref[1]) / deg + s_ref[...], 0.0)
    y_ref[...] = _dotT(h, wl_ref[...])
    self_ref[...] = _dotT(h, wr_ref[...]) + b_ref[...]


def _tc_mid(agg, d0T, d1T, selfp, Wl, Wr, b):
    return pl.pallas_call(
        _tcB_body,
        grid=(GRID,),
        in_specs=[
            pl.BlockSpec((NUM_SC, ROWS, D), lambda i: (0, i, 0)),
            pl.BlockSpec((ROWS, 1), lambda i: (i, 0)),
            pl.BlockSpec((ROWS, 1), lambda i: (i, 0)),
            pl.BlockSpec((ROWS, D), lambda i: (i, 0)),
            pl.BlockSpec((D, D), lambda i: (0, 0)),
            pl.BlockSpec((D, D), lambda i: (0, 0)),
            pl.BlockSpec((1, D), lambda i: (0, 0)),
        ],
        out_specs=[
            pl.BlockSpec((ROWS, D), lambda i: (i, 0)),
            pl.BlockSpec((ROWS, D), lambda i: (i, 0)),
        ],
        out_shape=[
            jax.ShapeDtypeStruct((N_NODES, D), jnp.float32),
            jax.ShapeDtypeStruct((N_NODES, D), jnp.float32),
        ],
    )(agg, d0T, d1T, selfp, Wl, Wr, b.reshape(1, D))


def _tcC_body(a_ref, d0_ref, d1_ref, s_ref, bt_ref, wlin_ref,
              blin_ref, out_ref, sums, cnts):
    i = pl.program_id(0)

    @pl.when(i == 0)
    def _():
        sums[...] = jnp.zeros_like(sums)
        cnts[...] = jnp.zeros_like(cnts)

    deg = jnp.maximum(d0_ref[...] + d1_ref[...], 1.0)
    h = jnp.maximum((a_ref[0] + a_ref[1]) / deg + s_ref[...], 0.0)
    bt = bt_ref[0]
    oh = (lax.broadcasted_iota(jnp.int32, (NUM_GRAPHS, ROWS), 0)
          == bt).astype(jnp.float32)
    sums[...] += jnp.dot(oh, h, preferred_element_type=jnp.float32)
    cnts[...] += jnp.sum(oh, axis=1, keepdims=True)

    @pl.when(i == pl.num_programs(0) - 1)
    def _():
        pooled = sums[...] / jnp.maximum(cnts[...], 1.0)
        out_ref[...] = _dotT(pooled, wlin_ref[...]) + blin_ref[...]


def _tc_post(agg, d0T, d1T, selfp, batch3, Wlin, blin):
    return pl.pallas_call(
        _tcC_body,
        grid=(GRID,),
        in_specs=[
            pl.BlockSpec((NUM_SC, ROWS, D), lambda i: (0, i, 0)),
            pl.BlockSpec((ROWS, 1), lambda i: (i, 0)),
            pl.BlockSpec((ROWS, 1), lambda i: (i, 0)),
            pl.BlockSpec((ROWS, D), lambda i: (i, 0)),
            pl.BlockSpec((1, 1, ROWS), lambda i: (i, 0, 0)),
            pl.BlockSpec((D_OUT, D), lambda i: (0, 0)),
            pl.BlockSpec((1, D_OUT), lambda i: (0, 0)),
        ],
        out_specs=pl.BlockSpec((NUM_GRAPHS, D_OUT), lambda i: (0, 0)),
        out_shape=jax.ShapeDtypeStruct((NUM_GRAPHS, D_OUT), jnp.float32),
        scratch_shapes=[
            pltpu.VMEM((NUM_GRAPHS, D), jnp.float32),
            pltpu.VMEM((NUM_GRAPHS, 1), jnp.float32),
        ],
    )(agg, d0T, d1T, selfp, batch3, Wlin, blin.reshape(1, D_OUT))


def _degcol(deg_slice):
    return deg_slice[:N_NODES].reshape(N_NODES, 1)


def kernel(x, edge_index, batch, W1l, b1, W1r, W2l, b2, W2r, Wlin, blin):
    E = edge_index.shape[1]
    src = edge_index[0]
    dst = edge_index[1]

    npad = E_PAD - E
    ar = np.arange(npad, dtype=np.int32)
    pad_src = jnp.asarray((ar * 997) % N_NODES, dtype=jnp.int32)
    pad_dst = jnp.asarray(N_NODES + ar % (N_PAD - N_NODES), dtype=jnp.int32)
    src2d = jnp.concatenate([src, pad_src]).reshape(E_PAD // CHUNK, CHUNK)
    dst2d = jnp.concatenate([dst, pad_dst]).reshape(E_PAD // CHUNK, CHUNK)

    zeros_blk = jnp.zeros((N_PAD, D), jnp.float32)
    zeros_deg = jnp.zeros((N_PAD,), jnp.float32)
    batch3 = batch.reshape(GRID, 1, ROWS)

    y1, self1 = _tc_pre(x, W1l, W1r, b1)
    agg1, deg = _sc_edge_pass(y1, src2d, dst2d, zeros_blk, zeros_deg,
                              with_deg=True)
    d0T = _degcol(deg[0])
    d1T = _degcol(deg[1])

    y2, self2 = _tc_mid(agg1, d0T, d1T, self1, W2l, W2r, b2)
    (agg2,) = _sc_edge_pass(y2, src2d, dst2d, zeros_blk, zeros_deg,
                            with_deg=False)

    return _tc_post(agg2, d0T, d1T, self2, batch3, Wlin, blin)


# speedup vs baseline: 1.0763x; 1.0619x over previous
import jax
import jax.numpy as jnp
import numpy as np
from jax import lax
from jax.experimental import pallas as pl
from jax.experimental.pallas import tpu as pltpu
from jax.experimental.pallas import tpu_sc as plsc

N_NODES = 10000
NUM_GRAPHS = 64
D = 128
D_OUT = 64

NUM_SC = 2
NUM_TILES = 16
CHUNK = 128
CPT = 80
NB = 2
NPHASE = 2
SPC = CPT // NPHASE
E_PAD = NUM_SC * NUM_TILES * CPT * CHUNK
REAL_CHUNKS = 320000 // CHUNK
PAD_CHUNKS = E_PAD // CHUNK - REAL_CHUNKS
N_PAD = 10240
ROWS_PER_TILE = N_PAD // NUM_TILES
ROWS = 2000
GRID = N_NODES // ROWS


def _sc_edge_pass(y, ei_pad, zeros_blk, zeros_deg, with_deg):
    mesh = plsc.VectorSubcoreMesh(
        core_axis_name="c", subcore_axis_name="s",
        num_cores=NUM_SC, num_subcores=NUM_TILES)

    out_type = [jax.ShapeDtypeStruct((NUM_SC, N_PAD, D), jnp.float32)]
    if with_deg:
        out_type.append(jax.ShapeDtypeStruct((NUM_SC, N_PAD), jnp.float32))

    scratch = [
        pltpu.VMEM_SHARED((N_PAD, D), jnp.float32),
        pltpu.VMEM((SPC, CHUNK), jnp.int32),
        pltpu.VMEM((SPC, CHUNK), jnp.int32),
        pltpu.VMEM((NB, CHUNK, D), jnp.float32),
        pltpu.SemaphoreType.DMA,
        pltpu.SemaphoreType.DMA,
        pltpu.SemaphoreType.DMA,
        pltpu.SemaphoreType.DMA,
    ]
    if with_deg:
        scratch += [
            pltpu.VMEM_SHARED((N_PAD,), jnp.float32),
            pltpu.VMEM((CHUNK,), jnp.float32),
            pltpu.SemaphoreType.DMA,
        ]

    def body(*refs):
        if with_deg:
            (y_hbm, ei_hbm, zblk_hbm, zdeg_hbm,
             agg_hbm, deg_hbm, accum, src_t, dst_t, rows, g0, g1, s0, s1,
             deg_sh, ones, sem_d) = refs
        else:
            (y_hbm, ei_hbm, zblk_hbm,
             agg_hbm, accum, src_t, dst_t, rows, g0, g1, s0, s1) = refs
        sem_g = (g0, g1)
        c = lax.axis_index("c")
        s = lax.axis_index("s")
        r0 = s * ROWS_PER_TILE

        zero_cp = pltpu.async_copy(
            zblk_hbm.at[pl.ds(r0, ROWS_PER_TILE)],
            accum.at[pl.ds(r0, ROWS_PER_TILE)], s0)
        if with_deg:
            dzero_cp = pltpu.async_copy(
                zdeg_hbm.at[pl.ds(r0, ROWS_PER_TILE)],
                deg_sh.at[pl.ds(r0, ROWS_PER_TILE)], s1)
            for j in range(CHUNK // 16):
                ones[pl.ds(j * 16, 16)] = jnp.full((16,), 1.0, jnp.float32)

        row_base = (c * NUM_TILES + s) * CPT
        for p in range(NPHASE):
            a = row_base + p * SPC
            pltpu.sync_copy(ei_hbm.at[0, pl.ds(a, SPC)], src_t)
            pltpu.sync_copy(ei_hbm.at[1, pl.ds(a, SPC)], dst_t)

            for b in range(NB):
                pltpu.async_copy(y_hbm.at[src_t.at[b]], rows.at[b], sem_g[b])

            if p == 0:
                zero_cp.wait()
                if with_deg:
                    dzero_cp.wait()
                plsc.subcore_barrier()

            def step(i, carry):
                for b in range(NB):
                    j = i * NB + b
                    idx_d = dst_t.at[j]
                    pltpu.make_async_copy(y_hbm.at[src_t.at[j]], rows.at[b],
                                          sem_g[b]).wait()
                    if with_deg:
                        pltpu.async_copy(ones, deg_sh.at[idx_d], sem_d,
                                         add=True)
                    pltpu.sync_copy(rows.at[b], accum.at[idx_d], add=True)

                    nj = j + NB

                    @pl.when(nj < SPC)
                    def _():
                        pltpu.async_copy(y_hbm.at[src_t.at[nj]], rows.at[b],
                                         sem_g[b])
                return carry

            lax.fori_loop(0, SPC // NB, step, 0)

            if with_deg:
                def drain(i, carry):
                    pltpu.make_async_copy(ones, deg_sh.at[dst_t.at[0]],
                                          sem_d).wait()
                    return carry
                lax.fori_loop(0, SPC, drain, 0)

        plsc.subcore_barrier()

        pltpu.sync_copy(accum.at[pl.ds(r0, ROWS_PER_TILE)],
                        agg_hbm.at[c, pl.ds(r0, ROWS_PER_TILE)])
        if with_deg:
            pltpu.sync_copy(deg_sh.at[pl.ds(r0, ROWS_PER_TILE)],
                            deg_hbm.at[c, pl.ds(r0, ROWS_PER_TILE)])

    run = pl.kernel(body, out_type=out_type, mesh=mesh, scratch_types=scratch)
    if with_deg:
        return run(y, ei_pad, zeros_blk, zeros_deg)
    return run(y, ei_pad, zeros_blk)


def _dotT(a, w):
    return lax.dot_general(a, w, (((1,), (1,)), ((), ())),
                           preferred_element_type=jnp.float32)


def _tcA_body(x_ref, wl_ref, wr_ref, b_ref, y_ref, self_ref):
    xb = x_ref[...]
    y_ref[...] = _dotT(xb, wl_ref[...])
    self_ref[...] = _dotT(xb, wr_ref[...]) + b_ref[...]


def _tc_pre(x, Wl, Wr, b):
    return pl.pallas_call(
        _tcA_body,
        grid=(GRID,),
        in_specs=[
            pl.BlockSpec((ROWS, D), lambda i: (i, 0)),
            pl.BlockSpec((D, D), lambda i: (0, 0)),
            pl.BlockSpec((D, D), lambda i: (0, 0)),
            pl.BlockSpec((1, D), lambda i: (0, 0)),
        ],
        out_specs=[
            pl.BlockSpec((ROWS, D), lambda i: (i, 0)),
            pl.BlockSpec((ROWS, D), lambda i: (i, 0)),
        ],
        out_shape=[
            jax.ShapeDtypeStruct((N_NODES, D), jnp.float32),
            jax.ShapeDtypeStruct((N_NODES, D), jnp.float32),
        ],
    )(x, Wl, Wr, b.reshape(1, D))


def _tcB_body(a_ref, d0_ref, d1_ref, s_ref, wl_ref, wr_ref, b_ref,
              y_ref, self_ref):
    deg = jnp.maximum(jnp.reshape(d0_ref[0] + d1_ref[0], (ROWS, 1)), 1.0)
    h = jnp.maximum((a_ref[0] + a_ref[1]) / deg + s_ref[...], 0.0)
    y_ref[...] = _dotT(h, wl_ref[...])
    self_ref[...] = _dotT(h, wr_ref[...]) + b_ref[...]


def _tc_mid(agg, d0T, d1T, selfp, Wl, Wr, b):
    return pl.pallas_call(
        _tcB_body,
        grid=(GRID,),
        in_specs=[
            pl.BlockSpec((NUM_SC, ROWS, D), lambda i: (0, i, 0)),
            pl.BlockSpec((1, 1, ROWS), lambda i: (i, 0, 0)),
            pl.BlockSpec((1, 1, ROWS), lambda i: (i, 0, 0)),
            pl.BlockSpec((ROWS, D), lambda i: (i, 0)),
            pl.BlockSpec((D, D), lambda i: (0, 0)),
            pl.BlockSpec((D, D), lambda i: (0, 0)),
            pl.BlockSpec((1, D), lambda i: (0, 0)),
        ],
        out_specs=[
            pl.BlockSpec((ROWS, D), lambda i: (i, 0)),
            pl.BlockSpec((ROWS, D), lambda i: (i, 0)),
        ],
        out_shape=[
            jax.ShapeDtypeStruct((N_NODES, D), jnp.float32),
            jax.ShapeDtypeStruct((N_NODES, D), jnp.float32),
        ],
    )(agg, d0T, d1T, selfp, Wl, Wr, b.reshape(1, D))


def _tcC_body(a_ref, d0_ref, d1_ref, s_ref, bt_ref, wlin_ref,
              blin_ref, out_ref, sums, cnts):
    i = pl.program_id(0)

    @pl.when(i == 0)
    def _():
        sums[...] = jnp.zeros_like(sums)
        cnts[...] = jnp.zeros_like(cnts)

    deg = jnp.maximum(jnp.reshape(d0_ref[0] + d1_ref[0], (ROWS, 1)), 1.0)
    h = jnp.maximum((a_ref[0] + a_ref[1]) / deg + s_ref[...], 0.0)
    bt = bt_ref[0]
    oh = (lax.broadcasted_iota(jnp.int32, (NUM_GRAPHS, ROWS), 0)
          == bt).astype(jnp.float32)
    sums[...] += jnp.dot(oh, h, preferred_element_type=jnp.float32)
    cnts[...] += jnp.sum(oh, axis=1, keepdims=True)

    @pl.when(i == pl.num_programs(0) - 1)
    def _():
        pooled = sums[...] / jnp.maximum(cnts[...], 1.0)
        out_ref[...] = _dotT(pooled, wlin_ref[...]) + blin_ref[...]


def _tc_post(agg, d0T, d1T, selfp, batch3, Wlin, blin):
    return pl.pallas_call(
        _tcC_body,
        grid=(GRID,),
        in_specs=[
            pl.BlockSpec((NUM_SC, ROWS, D), lambda i: (0, i, 0)),
            pl.BlockSpec((1, 1, ROWS), lambda i: (i, 0, 0)),
            pl.BlockSpec((1, 1, ROWS), lambda i: (i, 0, 0)),
            pl.BlockSpec((ROWS, D), lambda i: (i, 0)),
            pl.BlockSpec((1, 1, ROWS), lambda i: (i, 0, 0)),
            pl.BlockSpec((D_OUT, D), lambda i: (0, 0)),
            pl.BlockSpec((1, D_OUT), lambda i: (0, 0)),
        ],
        out_specs=pl.BlockSpec((NUM_GRAPHS, D_OUT), lambda i: (0, 0)),
        out_shape=jax.ShapeDtypeStruct((NUM_GRAPHS, D_OUT), jnp.float32),
        scratch_shapes=[
            pltpu.VMEM((NUM_GRAPHS, D), jnp.float32),
            pltpu.VMEM((NUM_GRAPHS, 1), jnp.float32),
        ],
    )(agg, d0T, d1T, selfp, batch3, Wlin, blin.reshape(1, D_OUT))


def _degrows(deg_slice):
    return deg_slice[:N_NODES].reshape(GRID, 1, ROWS)


def kernel(x, edge_index, batch, W1l, b1, W1r, W2l, b2, W2r, Wlin, blin):
    ar = np.arange(PAD_CHUNKS * CHUNK, dtype=np.int32)
    pad_blk = jnp.asarray(np.stack([
        ((ar * 997) % N_NODES).reshape(PAD_CHUNKS, CHUNK),
        (N_NODES + ar % (N_PAD - N_NODES)).reshape(PAD_CHUNKS, CHUNK),
    ]))
    ei_pad = jnp.concatenate(
        [edge_index.reshape(2, REAL_CHUNKS, CHUNK), pad_blk], axis=1)

    zeros_blk = jnp.asarray(np.zeros((N_PAD, D), np.float32))
    zeros_deg = jnp.asarray(np.zeros((N_PAD,), np.float32))
    batch3 = batch.reshape(GRID, 1, ROWS)

    y1, self1 = _tc_pre(x, W1l, W1r, b1)
    agg1, deg = _sc_edge_pass(y1, ei_pad, zeros_blk, zeros_deg,
                              with_deg=True)
    d0T = _degrows(deg[0])
    d1T = _degrows(deg[1])

    y2, self2 = _tc_mid(agg1, d0T, d1T, self1, W2l, W2r, b2)
    (agg2,) = _sc_edge_pass(y2, ei_pad, zeros_blk, zeros_deg,
                            with_deg=False)

    return _tc_post(agg2, d0T, d1T, self2, batch3, Wlin, blin)
